# baseline (speedup 1.0000x reference)
import functools

import jax
import jax.numpy as jnp
from jax import lax
from jax.experimental import pallas as pl
from jax.experimental.pallas import tpu as pltpu
from jax.experimental.pallas import tpu_sc as plsc

NC, NS, LANES = 2, 16, 16


def _deg_kernel(eidx, n_nodes):
    e = eidx.shape[1]
    nw = NC * NS
    ept = e // nw
    mesh = plsc.VectorSubcoreMesh(core_axis_name="c", subcore_axis_name="s")

    @functools.partial(
        pl.kernel,
        out_type=jax.ShapeDtypeStruct((nw, n_nodes), jnp.float32),
        mesh=mesh,
        scratch_types=[
            pltpu.VMEM((n_nodes,), jnp.float32),
            pltpu.VMEM((ept,), jnp.int32),
        ],
        compiler_params=pltpu.CompilerParams(needs_layout_passes=False),
    )
    def k(eidx_hbm, out_hbm, deg_l, dst_v):
        c = lax.axis_index("c")
        s = lax.axis_index("s")
        wid = c * NS + s

        def zero(i, _):
            deg_l[pl.ds(i * LANES, LANES)] = jnp.zeros((LANES,), jnp.float32)
            return 0

        lax.fori_loop(0, n_nodes // LANES, zero, 0)

        pltpu.sync_copy(eidx_hbm.at[1, pl.ds(wid * ept, ept)], dst_v)
        ones = jnp.ones((LANES,), jnp.float32)

        def acc(g, _):
            idx = dst_v[pl.ds(g * LANES, LANES)]
            plsc.addupdate_scatter(deg_l, [idx], ones)
            return 0

        lax.fori_loop(0, ept // LANES, acc, 0)
        pltpu.sync_copy(deg_l, out_hbm.at[wid])

    return k(eidx)


def _matmul_kernel(x, w):
    n, d_in = x.shape
    dh = w.shape[1]
    half = dh // 2
    blk = 1024

    def body(x_ref, w_ref, y_ref):
        y = jnp.dot(x_ref[...], w_ref[...], preferred_element_type=jnp.float32)
        y_ref[0] = y[:, :half]
        y_ref[1] = y[:, half:]

    return pl.pallas_call(
        body,
        grid=(n // blk,),
        in_specs=[
            pl.BlockSpec((blk, d_in), lambda i: (i, 0)),
            pl.BlockSpec((d_in, dh), lambda i: (0, 0)),
        ],
        out_specs=pl.BlockSpec((2, blk, half), lambda i: (0, i, 0)),
        out_shape=jax.ShapeDtypeStruct((2, n, half), jnp.float32),
    )(x, w)


def _scale_kernel(xw2, degp):
    _, n, half = xw2.shape
    blk = 1024
    nw = degp.shape[0]

    def body(xw_ref, degp_ref, y_ref):
        deg = jnp.sum(degp_ref[...], axis=0) + 1.0
        dinv = lax.rsqrt(deg)[:, None]
        y_ref[0] = xw_ref[0] * dinv
        y_ref[1] = xw_ref[1] * dinv

    return pl.pallas_call(
        body,
        grid=(n // blk,),
        in_specs=[
            pl.BlockSpec((2, blk, half), lambda i: (0, i, 0)),
            pl.BlockSpec((nw, blk), lambda i: (0, i)),
        ],
        out_specs=pl.BlockSpec((2, blk, half), lambda i: (0, i, 0)),
        out_shape=jax.ShapeDtypeStruct((2, n, half), jnp.float32),
    )(xw2, degp)


def _scatter_kernel(ycat, eidx, n_nodes):
    nrows = eidx.shape[1] // 128
    rpt = nrows // NS
    stripe = n_nodes // NS
    mesh = plsc.VectorSubcoreMesh(core_axis_name="c", subcore_axis_name="s")

    @functools.partial(
        pl.kernel,
        out_type=jax.ShapeDtypeStruct((2 * n_nodes, 128), jnp.float32),
        mesh=mesh,
        scratch_types=[
            pltpu.VMEM((rpt, 128), jnp.int32),
            pltpu.VMEM((rpt, 128), jnp.int32),
            pltpu.VMEM((128, 128), jnp.float32),
            pltpu.VMEM((128, 128), jnp.float32),
            pltpu.VMEM((128, 128), jnp.float32),
            pltpu.SemaphoreType.DMA,
            pltpu.SemaphoreType.DMA,
            pltpu.SemaphoreType.DMA,
            pltpu.SemaphoreType.DMA,
            pltpu.VMEM_SHARED((n_nodes, 128), jnp.float32),
        ],
    )
    def k(y_hbm, eidx_hbm, a_hbm,
          src_v, dst_v, buf0, buf1, buf2,
          gsem0, gsem1, ssem, seedsem, acc_sh):
        c = lax.axis_index("c")
        s = lax.axis_index("s")
        bufs = [buf0, buf1, buf2]
        gsems = [gsem0, gsem1]
        cbase = c * n_nodes
        ebase = s * (rpt * 128)

        seed = pltpu.async_copy(y_hbm.at[pl.ds(cbase + s * stripe, stripe)],
                                acc_sh.at[pl.ds(s * stripe, stripe)], seedsem)

        idx_d = []
        for r in range(rpt):
            idx_d.append(pltpu.async_copy(
                eidx_hbm.at[0, pl.ds(ebase + r * 128, 128)], src_v.at[r],
                gsem0))
            idx_d.append(pltpu.async_copy(
                eidx_hbm.at[1, pl.ds(ebase + r * 128, 128)], dst_v.at[r],
                gsem1))
        for d in idx_d:
            d.wait()

        def off(t, _):
            sl = (t // 8, pl.ds((t % 8) * LANES, LANES))
            src_v[sl] = src_v[sl] + cbase
            return 0

        lax.fori_loop(0, rpt * 8, off, 0)

        gd = [None] * rpt
        sd = [None] * rpt
        gd[0] = pltpu.async_copy(y_hbm.at[src_v.at[0]], bufs[0], gsems[0])
        if rpt > 1:
            gd[1] = pltpu.async_copy(y_hbm.at[src_v.at[1]], bufs[1], gsems[1])
        seed.wait()
        plsc.subcore_barrier()
        for j in range(rpt):
            gd[j].wait()
            if j >= 1:
                sd[j - 1].wait()
            sd[j] = pltpu.async_copy(bufs[j % 3], acc_sh.at[dst_v.at[j]],
                                     ssem, add=True)
            if j + 2 < rpt:
                gd[j + 2] = pltpu.async_copy(
                    y_hbm.at[src_v.at[j + 2]], bufs[(j + 2) % 3],
                    gsems[j % 2])
        sd[rpt - 1].wait()
        plsc.subcore_barrier()
        pltpu.sync_copy(acc_sh.at[pl.ds(s * stripe, stripe)],
                        a_hbm.at[pl.ds(cbase + s * stripe, stripe)])

    return k(ycat, eidx)


def _bn_fc_kernel(acat, degp, bc2, gamma1, beta1, fc_w, fcb1):
    n2, half = acat.shape
    n = n2 // 2
    dh = 2 * half
    d4 = 4 * dh
    blk = 512
    gblk = 256
    g_all = n // 4
    p1 = n // blk
    p2 = g_all // gblk
    nw = degp.shape[0]
    ncls = fc_w.shape[0]
    inv_n = 1.0 / float(n)

    def body(a0_ref, a1_ref, degp_ref, bc_ref, ga_ref, be_ref, fw_ref,
             fb_ref, out_ref, hg_s, st_s):
        i = pl.program_id(0)

        @pl.when(i == 0)
        def _():
            st_s[...] = jnp.zeros_like(st_s)

        @pl.when(i < p1)
        def _():
            deg = jnp.sum(degp_ref[...], axis=0) + 1.0
            dinv = lax.rsqrt(deg)[:, None]
            h0 = jnp.maximum(dinv * a0_ref[...] + bc_ref[0:1, :], 0.0)
            h1 = jnp.maximum(dinv * a1_ref[...] + bc_ref[1:2, :], 0.0)
            hcat = jnp.concatenate([h0, h1], axis=1)
            hg_s[pl.ds(i * (blk // 4), blk // 4), :] = hcat.reshape(
                blk // 4, d4)
            row_s = jnp.concatenate(
                [jnp.sum(h0, axis=0), jnp.sum(h1, axis=0)])
            row_q = jnp.concatenate(
                [jnp.sum(h0 * h0, axis=0), jnp.sum(h1 * h1, axis=0)])
            st_s[0:1, :] += row_s[None, :]
            st_s[1:2, :] += row_q[None, :]

        @pl.when(i >= p1)
        def _():
            j = i - p1
            mean = st_s[0:1, :] * inv_n
            ex2 = st_s[1:2, :] * inv_n
            var = ex2 - mean * mean
            rstd = lax.rsqrt(var + 1e-5)
            scale = ga_ref[0:1, :] * rstd
            off = be_ref[0:1, :] - mean * scale
            scale4 = jnp.concatenate([scale] * 4, axis=1)
            off4 = jnp.concatenate([off] * 4, axis=1)
            hgn = hg_s[pl.ds(j * gblk, gblk), :] * scale4 + off4
            ltT = lax.dot_general(
                fw_ref[...], hgn, (((1,), (1,)), ((), ())),
                preferred_element_type=jnp.float32) + fb_ref[...]
            m = jnp.max(ltT, axis=0, keepdims=True)
            lse = m + jnp.log(
                jnp.sum(jnp.exp(ltT - m), axis=0, keepdims=True))
            lsT = ltT - lse
            gsrc = lax.broadcasted_iota(jnp.int32, (gblk, 4 * gblk), 0)
            gdst = lax.broadcasted_iota(jnp.int32, (gblk, 4 * gblk), 1) // 4
            rep = (gsrc == gdst).astype(jnp.float32)
            out_ref[...] = lax.dot_general(
                lsT, rep, (((1,), (0,)), ((), ())),
                preferred_element_type=jnp.float32)

    return pl.pallas_call(
        body,
        grid=(p1 + p2,),
        in_specs=[
            pl.BlockSpec((blk, half), lambda i: (jnp.minimum(i, p1 - 1), 0)),
            pl.BlockSpec((blk, half),
                         lambda i: (p1 + jnp.minimum(i, p1 - 1), 0)),
            pl.BlockSpec((nw, blk), lambda i: (0, jnp.minimum(i, p1 - 1))),
            pl.BlockSpec((2, half), lambda i: (0, 0)),
            pl.BlockSpec((1, dh), lambda i: (0, 0)),
            pl.BlockSpec((1, dh), lambda i: (0, 0)),
            pl.BlockSpec((ncls, d4), lambda i: (0, 0)),
            pl.BlockSpec((ncls, 1), lambda i: (0, 0)),
        ],
        out_specs=pl.BlockSpec(
            (ncls, 4 * gblk), lambda i: (0, jnp.maximum(i - p1, 0))),
        out_shape=jax.ShapeDtypeStruct((ncls, n), jnp.float32),
        scratch_shapes=[
            pltpu.VMEM((g_all, d4), jnp.float32),
            pltpu.VMEM((8, dh), jnp.float32),
        ],
    )(acat, acat, degp, bc2, gamma1, beta1, fc_w, fcb1)


def kernel(x, edge_index, num_graphs, W, b_conv, gamma, beta, fc_W, fc_b):
    del num_graphs
    n, _ = x.shape
    dh = W.shape[1]
    degp = _deg_kernel(edge_index, n)
    xw2 = _matmul_kernel(x, W)
    ycat = _scale_kernel(xw2, degp).reshape(2 * n, dh // 2)
    acat = _scatter_kernel(ycat, edge_index, n)
    bc2 = b_conv.reshape(2, dh // 2)
    outT = _bn_fc_kernel(acat, degp, bc2, gamma.reshape(1, dh),
                         beta.reshape(1, dh), fc_W, fc_b.reshape(-1, 1))
    return outT.T


# speedup vs baseline: 1.0196x; 1.0196x over previous
import functools

import jax
import jax.numpy as jnp
from jax import lax
from jax.experimental import pallas as pl
from jax.experimental.pallas import tpu as pltpu
from jax.experimental.pallas import tpu_sc as plsc

NC, NS, LANES = 2, 16, 16


def _deg_kernel(eidx, n_nodes):
    e = eidx.shape[1]
    nw = NC * NS
    ept = e // nw
    mesh = plsc.VectorSubcoreMesh(core_axis_name="c", subcore_axis_name="s")

    @functools.partial(
        pl.kernel,
        out_type=jax.ShapeDtypeStruct((nw, n_nodes), jnp.float32),
        mesh=mesh,
        scratch_types=[
            pltpu.VMEM((n_nodes,), jnp.float32),
            pltpu.VMEM((ept,), jnp.int32),
        ],
        compiler_params=pltpu.CompilerParams(needs_layout_passes=False),
    )
    def k(eidx_hbm, out_hbm, deg_l, dst_v):
        c = lax.axis_index("c")
        s = lax.axis_index("s")
        wid = c * NS + s

        def zero(i, _):
            deg_l[pl.ds(i * LANES, LANES)] = jnp.zeros((LANES,), jnp.float32)
            return 0

        lax.fori_loop(0, n_nodes // LANES, zero, 0)

        pltpu.sync_copy(eidx_hbm.at[1, pl.ds(wid * ept, ept)], dst_v)
        ones = jnp.ones((LANES,), jnp.float32)

        def acc(g, _):
            idx = dst_v[pl.ds(g * LANES, LANES)]
            plsc.addupdate_scatter(deg_l, [idx], ones)
            return 0

        lax.fori_loop(0, ept // LANES, acc, 0)
        pltpu.sync_copy(deg_l, out_hbm.at[wid])

    return k(eidx)


def _xw_top_kernel(x, w):
    n, d_in = x.shape
    dh = w.shape[1]
    half = dh // 2
    blk = 1024
    base = n // (2 * blk)

    def body(x_ref, w_ref, y_ref):
        y = jnp.dot(x_ref[...], w_ref[...], preferred_element_type=jnp.float32)
        y_ref[0] = y[:, :half]
        y_ref[1] = y[:, half:]

    return pl.pallas_call(
        body,
        grid=(base,),
        in_specs=[
            pl.BlockSpec((blk, d_in), lambda i: (base + i, 0)),
            pl.BlockSpec((d_in, dh), lambda i: (0, 0)),
        ],
        out_specs=pl.BlockSpec((2, blk, half), lambda i: (0, i, 0)),
        out_shape=jax.ShapeDtypeStruct((2, n // 2, half), jnp.float32),
    )(x, w)


def _xw_scale_kernel(x, w, degp, xwtop):
    n, d_in = x.shape
    dh = w.shape[1]
    half = dh // 2
    blk = 1024
    p = n // (2 * blk)
    nw = degp.shape[0]

    def body(x_ref, w_ref, degp_ref, xt_ref, y_ref):
        i = pl.program_id(0)
        deg = jnp.sum(degp_ref[...], axis=0) + 1.0
        dinv = lax.rsqrt(deg)[:, None]

        @pl.when(i < p)
        def _():
            y = jnp.dot(x_ref[...], w_ref[...],
                        preferred_element_type=jnp.float32)
            y_ref[0] = y[:, :half] * dinv
            y_ref[1] = y[:, half:] * dinv

        @pl.when(i >= p)
        def _():
            y_ref[0] = xt_ref[0] * dinv
            y_ref[1] = xt_ref[1] * dinv

    return pl.pallas_call(
        body,
        grid=(2 * p,),
        in_specs=[
            pl.BlockSpec((blk, d_in), lambda i: (jnp.minimum(i, p - 1), 0)),
            pl.BlockSpec((d_in, dh), lambda i: (0, 0)),
            pl.BlockSpec((nw, blk), lambda i: (0, i)),
            pl.BlockSpec((2, blk, half),
                         lambda i: (0, jnp.maximum(i - p, 0), 0)),
        ],
        out_specs=pl.BlockSpec((2, blk, half), lambda i: (0, i, 0)),
        out_shape=jax.ShapeDtypeStruct((2, n, half), jnp.float32),
    )(x, w, degp, xwtop)


def _scatter_kernel(ycat, eidx, n_nodes):
    nrows = eidx.shape[1] // 128
    rpt = nrows // NS
    stripe = n_nodes // NS
    mesh = plsc.VectorSubcoreMesh(core_axis_name="c", subcore_axis_name="s")

    @functools.partial(
        pl.kernel,
        out_type=jax.ShapeDtypeStruct((2 * n_nodes, 128), jnp.float32),
        mesh=mesh,
        scratch_types=[
            pltpu.VMEM((rpt, 128), jnp.int32),
            pltpu.VMEM((rpt, 128), jnp.int32),
            pltpu.VMEM((128, 128), jnp.float32),
            pltpu.VMEM((128, 128), jnp.float32),
            pltpu.VMEM((128, 128), jnp.float32),
            pltpu.SemaphoreType.DMA,
            pltpu.SemaphoreType.DMA,
            pltpu.SemaphoreType.DMA,
            pltpu.SemaphoreType.DMA,
            pltpu.VMEM_SHARED((n_nodes, 128), jnp.float32),
        ],
    )
    def k(y_hbm, eidx_hbm, a_hbm,
          src_v, dst_v, buf0, buf1, buf2,
          gsem0, gsem1, ssem, seedsem, acc_sh):
        c = lax.axis_index("c")
        s = lax.axis_index("s")
        bufs = [buf0, buf1, buf2]
        gsems = [gsem0, gsem1]
        cbase = c * n_nodes
        ebase = s * (rpt * 128)

        seed = pltpu.async_copy(y_hbm.at[pl.ds(cbase + s * stripe, stripe)],
                                acc_sh.at[pl.ds(s * stripe, stripe)], seedsem)

        idx_d = []
        for r in range(rpt):
            idx_d.append(pltpu.async_copy(
                eidx_hbm.at[0, pl.ds(ebase + r * 128, 128)], src_v.at[r],
                gsem0))
            idx_d.append(pltpu.async_copy(
                eidx_hbm.at[1, pl.ds(ebase + r * 128, 128)], dst_v.at[r],
                gsem1))
        for d in idx_d:
            d.wait()

        def off(t, _):
            sl = (t // 8, pl.ds((t % 8) * LANES, LANES))
            src_v[sl] = src_v[sl] + cbase
            return 0

        lax.fori_loop(0, rpt * 8, off, 0)

        gd = [None] * rpt
        sd = [None] * rpt
        gd[0] = pltpu.async_copy(y_hbm.at[src_v.at[0]], bufs[0], gsems[0])
        if rpt > 1:
            gd[1] = pltpu.async_copy(y_hbm.at[src_v.at[1]], bufs[1], gsems[1])
        seed.wait()
        plsc.subcore_barrier()
        for j in range(rpt):
            gd[j].wait()
            if j >= 1:
                sd[j - 1].wait()
            sd[j] = pltpu.async_copy(bufs[j % 3], acc_sh.at[dst_v.at[j]],
                                     ssem, add=True)
            if j + 2 < rpt:
                gd[j + 2] = pltpu.async_copy(
                    y_hbm.at[src_v.at[j + 2]], bufs[(j + 2) % 3],
                    gsems[j % 2])
        sd[rpt - 1].wait()
        plsc.subcore_barrier()
        pltpu.sync_copy(acc_sh.at[pl.ds(s * stripe, stripe)],
                        a_hbm.at[pl.ds(cbase + s * stripe, stripe)])

    return k(ycat, eidx)


def _bn_fc_kernel(acat, degp, bc2, gamma1, beta1, fc_w, fcb1):
    n2, half = acat.shape
    n = n2 // 2
    dh = 2 * half
    d4 = 4 * dh
    blk = 512
    gblk = 256
    g_all = n // 4
    p1 = n // blk
    p2 = g_all // gblk
    nw = degp.shape[0]
    ncls = fc_w.shape[0]
    inv_n = 1.0 / float(n)

    def body(a0_ref, a1_ref, degp_ref, bc_ref, ga_ref, be_ref, fw_ref,
             fb_ref, out_ref, hg_s, st_s):
        i = pl.program_id(0)

        @pl.when(i == 0)
        def _():
            st_s[...] = jnp.zeros_like(st_s)

        @pl.when(i < p1)
        def _():
            deg = jnp.sum(degp_ref[...], axis=0) + 1.0
            dinv = lax.rsqrt(deg)[:, None]
            h0 = jnp.maximum(dinv * a0_ref[...] + bc_ref[0:1, :], 0.0)
            h1 = jnp.maximum(dinv * a1_ref[...] + bc_ref[1:2, :], 0.0)
            hcat = jnp.concatenate([h0, h1], axis=1)
            hg_s[pl.ds(i * (blk // 4), blk // 4), :] = hcat.reshape(
                blk // 4, d4)
            row_s = jnp.concatenate(
                [jnp.sum(h0, axis=0), jnp.sum(h1, axis=0)])
            row_q = jnp.concatenate(
                [jnp.sum(h0 * h0, axis=0), jnp.sum(h1 * h1, axis=0)])
            st_s[0:1, :] += row_s[None, :]
            st_s[1:2, :] += row_q[None, :]

        @pl.when(i >= p1)
        def _():
            j = i - p1
            mean = st_s[0:1, :] * inv_n
            ex2 = st_s[1:2, :] * inv_n
            var = ex2 - mean * mean
            rstd = lax.rsqrt(var + 1e-5)
            scale = ga_ref[0:1, :] * rstd
            off = be_ref[0:1, :] - mean * scale
            scale4 = jnp.concatenate([scale] * 4, axis=1)
            off4 = jnp.concatenate([off] * 4, axis=1)
            hgn = hg_s[pl.ds(j * gblk, gblk), :] * scale4 + off4
            ltT = lax.dot_general(
                fw_ref[...], hgn, (((1,), (1,)), ((), ())),
                preferred_element_type=jnp.float32) + fb_ref[...]
            m = jnp.max(ltT, axis=0, keepdims=True)
            lse = m + jnp.log(
                jnp.sum(jnp.exp(ltT - m), axis=0, keepdims=True))
            lsT = ltT - lse
            gsrc = lax.broadcasted_iota(jnp.int32, (gblk, 4 * gblk), 0)
            gdst = lax.broadcasted_iota(jnp.int32, (gblk, 4 * gblk), 1) // 4
            rep = (gsrc == gdst).astype(jnp.float32)
            out_ref[...] = lax.dot_general(
                lsT, rep, (((1,), (0,)), ((), ())),
                preferred_element_type=jnp.float32)

    return pl.pallas_call(
        body,
        grid=(p1 + p2,),
        in_specs=[
            pl.BlockSpec((blk, half), lambda i: (jnp.minimum(i, p1 - 1), 0)),
            pl.BlockSpec((blk, half),
                         lambda i: (p1 + jnp.minimum(i, p1 - 1), 0)),
            pl.BlockSpec((nw, blk), lambda i: (0, jnp.minimum(i, p1 - 1))),
            pl.BlockSpec((2, half), lambda i: (0, 0)),
            pl.BlockSpec((1, dh), lambda i: (0, 0)),
            pl.BlockSpec((1, dh), lambda i: (0, 0)),
            pl.BlockSpec((ncls, d4), lambda i: (0, 0)),
            pl.BlockSpec((ncls, 1), lambda i: (0, 0)),
        ],
        out_specs=pl.BlockSpec(
            (ncls, 4 * gblk), lambda i: (0, jnp.maximum(i - p1, 0))),
        out_shape=jax.ShapeDtypeStruct((ncls, n), jnp.float32),
        scratch_shapes=[
            pltpu.VMEM((g_all, d4), jnp.float32),
            pltpu.VMEM((8, dh), jnp.float32),
        ],
    )(acat, acat, degp, bc2, gamma1, beta1, fc_w, fcb1)


def kernel(x, edge_index, num_graphs, W, b_conv, gamma, beta, fc_W, fc_b):
    del num_graphs
    n, _ = x.shape
    dh = W.shape[1]
    degp = _deg_kernel(edge_index, n)
    xwtop = _xw_top_kernel(x, W)
    ycat = _xw_scale_kernel(x, W, degp, xwtop).reshape(2 * n, dh // 2)
    acat = _scatter_kernel(ycat, edge_index, n)
    bc2 = b_conv.reshape(2, dh // 2)
    outT = _bn_fc_kernel(acat, degp, bc2, gamma.reshape(1, dh),
                         beta.reshape(1, dh), fc_W, fc_b.reshape(-1, 1))
    return outT.T


# speedup vs baseline: 1.0947x; 1.0737x over previous
import functools

import jax
import jax.numpy as jnp
from jax import lax
from jax.experimental import pallas as pl
from jax.experimental.pallas import tpu as pltpu
from jax.experimental.pallas import tpu_sc as plsc

NC, NS, LANES = 2, 16, 16


def _deg_kernel(eidx, n_nodes):
    e = eidx.shape[1]
    nw = NC * NS
    ept = e // nw
    mesh = plsc.VectorSubcoreMesh(core_axis_name="c", subcore_axis_name="s")

    @functools.partial(
        pl.kernel,
        out_type=jax.ShapeDtypeStruct((nw, n_nodes), jnp.float32),
        mesh=mesh,
        scratch_types=[
            pltpu.VMEM((n_nodes,), jnp.float32),
            pltpu.VMEM((ept,), jnp.int32),
        ],
        compiler_params=pltpu.CompilerParams(needs_layout_passes=False),
    )
    def k(eidx_hbm, out_hbm, deg_l, dst_v):
        c = lax.axis_index("c")
        s = lax.axis_index("s")
        wid = c * NS + s

        def zero(i, _):
            deg_l[pl.ds(i * LANES, LANES)] = jnp.zeros((LANES,), jnp.float32)
            return 0

        lax.fori_loop(0, n_nodes // LANES, zero, 0)

        pltpu.sync_copy(eidx_hbm.at[1, pl.ds(wid * ept, ept)], dst_v)
        ones = jnp.ones((LANES,), jnp.float32)

        def acc(g, _):
            idx = dst_v[pl.ds(g * LANES, LANES)]
            plsc.addupdate_scatter(deg_l, [idx], ones)
            return 0

        lax.fori_loop(0, ept // LANES, acc, 0)
        pltpu.sync_copy(deg_l, out_hbm.at[wid])

    return k(eidx)


_TOP_BLOCKS = 3


def _xw_top_kernel(x, w):
    n, d_in = x.shape
    dh = w.shape[1]
    half = dh // 2
    blk = 1024
    base = n // blk - _TOP_BLOCKS

    def body(x_ref, w_ref, y_ref):
        y = jnp.dot(x_ref[...], w_ref[...], preferred_element_type=jnp.float32)
        y_ref[0] = y[:, :half]
        y_ref[1] = y[:, half:]

    return pl.pallas_call(
        body,
        grid=(_TOP_BLOCKS,),
        in_specs=[
            pl.BlockSpec((blk, d_in), lambda i: (base + i, 0)),
            pl.BlockSpec((d_in, dh), lambda i: (0, 0)),
        ],
        out_specs=pl.BlockSpec((2, blk, half), lambda i: (0, i, 0)),
        out_shape=jax.ShapeDtypeStruct((2, _TOP_BLOCKS * blk, half),
                                       jnp.float32),
    )(x, w)


def _xw_scale_kernel(x, w, degp, xwtop):
    n, d_in = x.shape
    dh = w.shape[1]
    half = dh // 2
    blk = 1024
    pt = _TOP_BLOCKS
    p = n // blk - pt
    nw = degp.shape[0]

    def body(x_ref, w_ref, degp_ref, xt_ref, y_ref):
        i = pl.program_id(0)
        deg = jnp.sum(degp_ref[...], axis=0) + 1.0
        dinv = lax.rsqrt(deg)[:, None]

        @pl.when(i < p)
        def _():
            y = jnp.dot(x_ref[...], w_ref[...],
                        preferred_element_type=jnp.float32)
            y_ref[0] = y[:, :half] * dinv
            y_ref[1] = y[:, half:] * dinv

        @pl.when(i >= p)
        def _():
            y_ref[0] = xt_ref[0] * dinv
            y_ref[1] = xt_ref[1] * dinv

    return pl.pallas_call(
        body,
        grid=(p + pt,),
        in_specs=[
            pl.BlockSpec((blk, d_in), lambda i: (jnp.minimum(i, p - 1), 0)),
            pl.BlockSpec((d_in, dh), lambda i: (0, 0)),
            pl.BlockSpec((nw, blk), lambda i: (0, i)),
            pl.BlockSpec((2, blk, half),
                         lambda i: (0, jnp.maximum(i - p, 0), 0)),
        ],
        out_specs=pl.BlockSpec((2, blk, half), lambda i: (0, i, 0)),
        out_shape=jax.ShapeDtypeStruct((2, n, half), jnp.float32),
    )(x, w, degp, xwtop)


def _scatter_kernel(ycat, eidx, n_nodes):
    nrows = eidx.shape[1] // 128
    rpt = nrows // NS
    stripe = n_nodes // NS
    mesh = plsc.VectorSubcoreMesh(core_axis_name="c", subcore_axis_name="s")

    @functools.partial(
        pl.kernel,
        out_type=jax.ShapeDtypeStruct((2 * n_nodes, 128), jnp.float32),
        mesh=mesh,
        scratch_types=[
            pltpu.VMEM((rpt, 128), jnp.int32),
            pltpu.VMEM((rpt, 128), jnp.int32),
            pltpu.VMEM((128, 128), jnp.float32),
            pltpu.VMEM((128, 128), jnp.float32),
            pltpu.VMEM((128, 128), jnp.float32),
            pltpu.SemaphoreType.DMA,
            pltpu.SemaphoreType.DMA,
            pltpu.SemaphoreType.DMA,
            pltpu.SemaphoreType.DMA,
            pltpu.VMEM_SHARED((n_nodes, 128), jnp.float32),
        ],
    )
    def k(y_hbm, eidx_hbm, a_hbm,
          src_v, dst_v, buf0, buf1, buf2,
          gsem0, gsem1, ssem, seedsem, acc_sh):
        c = lax.axis_index("c")
        s = lax.axis_index("s")
        bufs = [buf0, buf1, buf2]
        gsems = [gsem0, gsem1]
        cbase = c * n_nodes
        ebase = s * (rpt * 128)

        seed = pltpu.async_copy(y_hbm.at[pl.ds(cbase + s * stripe, stripe)],
                                acc_sh.at[pl.ds(s * stripe, stripe)], seedsem)

        idx_d = []
        for r in range(rpt):
            idx_d.append(pltpu.async_copy(
                eidx_hbm.at[0, pl.ds(ebase + r * 128, 128)], src_v.at[r],
                gsem0))
            idx_d.append(pltpu.async_copy(
                eidx_hbm.at[1, pl.ds(ebase + r * 128, 128)], dst_v.at[r],
                gsem1))
        for d in idx_d:
            d.wait()

        def off(t, _):
            sl = (t // 8, pl.ds((t % 8) * LANES, LANES))
            src_v[sl] = src_v[sl] + cbase
            return 0

        lax.fori_loop(0, rpt * 8, off, 0)

        gd = [None] * rpt
        sd = [None] * rpt
        gd[0] = pltpu.async_copy(y_hbm.at[src_v.at[0]], bufs[0], gsems[0])
        if rpt > 1:
            gd[1] = pltpu.async_copy(y_hbm.at[src_v.at[1]], bufs[1], gsems[1])
        seed.wait()
        plsc.subcore_barrier()
        for j in range(rpt):
            gd[j].wait()
            if j >= 1:
                sd[j - 1].wait()
            sd[j] = pltpu.async_copy(bufs[j % 3], acc_sh.at[dst_v.at[j]],
                                     ssem, add=True)
            if j + 2 < rpt:
                gd[j + 2] = pltpu.async_copy(
                    y_hbm.at[src_v.at[j + 2]], bufs[(j + 2) % 3],
                    gsems[j % 2])
        sd[rpt - 1].wait()
        plsc.subcore_barrier()
        pltpu.sync_copy(acc_sh.at[pl.ds(s * stripe, stripe)],
                        a_hbm.at[pl.ds(cbase + s * stripe, stripe)])

    return k(ycat, eidx)


def _bn_fc_kernel(acat, degp, bc2, gamma1, beta1, fc_w, fcb1):
    n2, half = acat.shape
    n = n2 // 2
    dh = 2 * half
    d4 = 4 * dh
    blk = 1024
    gblk = 512
    g_all = n // 4
    p1 = n // blk
    p2 = g_all // gblk
    nw = degp.shape[0]
    ncls = fc_w.shape[0]
    inv_n = 1.0 / float(n)

    def body(a0_ref, a1_ref, degp_ref, bc_ref, ga_ref, be_ref, fw_ref,
             fb_ref, out_ref, hg_s, st_s):
        i = pl.program_id(0)

        @pl.when(i == 0)
        def _():
            st_s[...] = jnp.zeros_like(st_s)

        @pl.when(i < p1)
        def _():
            deg = jnp.sum(degp_ref[...], axis=0) + 1.0
            dinv = lax.rsqrt(deg)[:, None]
            h0 = jnp.maximum(dinv * a0_ref[...] + bc_ref[0:1, :], 0.0)
            h1 = jnp.maximum(dinv * a1_ref[...] + bc_ref[1:2, :], 0.0)
            hcat = jnp.concatenate([h0, h1], axis=1)
            hg_s[pl.ds(i * (blk // 4), blk // 4), :] = hcat.reshape(
                blk // 4, d4)
            row_s = jnp.concatenate(
                [jnp.sum(h0, axis=0), jnp.sum(h1, axis=0)])
            row_q = jnp.concatenate(
                [jnp.sum(h0 * h0, axis=0), jnp.sum(h1 * h1, axis=0)])
            st_s[0:1, :] += row_s[None, :]
            st_s[1:2, :] += row_q[None, :]

        @pl.when(i >= p1)
        def _():
            j = i - p1
            mean = st_s[0:1, :] * inv_n
            ex2 = st_s[1:2, :] * inv_n
            var = ex2 - mean * mean
            rstd = lax.rsqrt(var + 1e-5)
            scale = ga_ref[0:1, :] * rstd
            off = be_ref[0:1, :] - mean * scale
            scale4 = jnp.concatenate([scale] * 4, axis=1)
            off4 = jnp.concatenate([off] * 4, axis=1)
            hgn = hg_s[pl.ds(j * gblk, gblk), :] * scale4 + off4
            ltT = lax.dot_general(
                fw_ref[...], hgn, (((1,), (1,)), ((), ())),
                preferred_element_type=jnp.float32) + fb_ref[...]
            m = jnp.max(ltT, axis=0, keepdims=True)
            lse = m + jnp.log(
                jnp.sum(jnp.exp(ltT - m), axis=0, keepdims=True))
            lsT = ltT - lse
            gsrc = lax.broadcasted_iota(jnp.int32, (gblk, 4 * gblk), 0)
            gdst = lax.broadcasted_iota(jnp.int32, (gblk, 4 * gblk), 1) // 4
            rep = (gsrc == gdst).astype(jnp.float32)
            out_ref[...] = lax.dot_general(
                lsT, rep, (((1,), (0,)), ((), ())),
                preferred_element_type=jnp.float32)

    return pl.pallas_call(
        body,
        grid=(p1 + p2,),
        in_specs=[
            pl.BlockSpec((blk, half), lambda i: (jnp.minimum(i, p1 - 1), 0)),
            pl.BlockSpec((blk, half),
                         lambda i: (p1 + jnp.minimum(i, p1 - 1), 0)),
            pl.BlockSpec((nw, blk), lambda i: (0, jnp.minimum(i, p1 - 1))),
            pl.BlockSpec((2, half), lambda i: (0, 0)),
            pl.BlockSpec((1, dh), lambda i: (0, 0)),
            pl.BlockSpec((1, dh), lambda i: (0, 0)),
            pl.BlockSpec((ncls, d4), lambda i: (0, 0)),
            pl.BlockSpec((ncls, 1), lambda i: (0, 0)),
        ],
        out_specs=pl.BlockSpec(
            (ncls, 4 * gblk), lambda i: (0, jnp.maximum(i - p1, 0))),
        out_shape=jax.ShapeDtypeStruct((ncls, n), jnp.float32),
        scratch_shapes=[
            pltpu.VMEM((g_all, d4), jnp.float32),
            pltpu.VMEM((8, dh), jnp.float32),
        ],
    )(acat, acat, degp, bc2, gamma1, beta1, fc_w, fcb1)


def kernel(x, edge_index, num_graphs, W, b_conv, gamma, beta, fc_W, fc_b):
    del num_graphs
    n, _ = x.shape
    dh = W.shape[1]
    degp = _deg_kernel(edge_index, n)
    xwtop = _xw_top_kernel(x, W)
    ycat = _xw_scale_kernel(x, W, degp, xwtop).reshape(2 * n, dh // 2)
    acat = _scatter_kernel(ycat, edge_index, n)
    bc2 = b_conv.reshape(2, dh // 2)
    outT = _bn_fc_kernel(acat, degp, bc2, gamma.reshape(1, dh),
                         beta.reshape(1, dh), fc_W, fc_b.reshape(-1, 1))
    return outT.T
